# Initial kernel scaffold; baseline (speedup 1.0000x reference)
import functools

import jax
import jax.numpy as jnp
from jax import lax
from jax.experimental import pallas as pl
from jax.experimental.pallas import tpu as pltpu
from jax.experimental.pallas import tpu_sc as plsc

D = 64
B = 16384 * 50
NW = 32
BPW = B // NW
CH = 128
NCH = BPW // CH
NBUF = 2

_mesh = plsc.VectorSubcoreMesh(core_axis_name="c", subcore_axis_name="s")


@functools.partial(
    pl.kernel,
    mesh=_mesh,
    out_type=jax.ShapeDtypeStruct((2, B, D), jnp.float32),
    scratch_types=[
        pltpu.VMEM((NCH, CH), jnp.int32),
        pltpu.VMEM((NBUF, CH, D), jnp.float32),
        pltpu.SemaphoreType.DMA,
        pltpu.SemaphoreType.DMA,
    ],
)
def _emb_lookup(src_w, tgt_w, src_idx, tgt_idx, out, idx_v, rows_v, gsem, ssem):
    wid = lax.axis_index("s") * 2 + lax.axis_index("c")
    base = wid * BPW
    for t in range(2):
        table = src_w if t == 0 else tgt_w
        idx_hbm = src_idx if t == 0 else tgt_idx
        pltpu.sync_copy(idx_hbm.at[wid], idx_v)

        pltpu.async_copy(table.at[idx_v.at[0]], rows_v.at[0], gsem)

        def chunk_body(j, _):
            slot = lax.rem(j, NBUF)
            nslot = lax.rem(j + 1, NBUF)

            pltpu.make_async_copy(
                table.at[idx_v.at[j]], rows_v.at[slot], gsem
            ).wait()
            pltpu.async_copy(
                rows_v.at[slot], out.at[t, pl.ds(base + j * CH, CH)], ssem
            )

            @pl.when(j >= 1)
            def _():
                pltpu.make_async_copy(
                    rows_v.at[nslot],
                    out.at[t, pl.ds(base + (j - 1) * CH, CH)],
                    ssem,
                ).wait()

            @pl.when(j + 1 < NCH)
            def _():
                pltpu.async_copy(
                    table.at[idx_v.at[j + 1]], rows_v.at[nslot], gsem
                )

            return 0

        lax.fori_loop(0, NCH, chunk_body, 0)
        pltpu.make_async_copy(
            rows_v.at[(NCH - 1) % NBUF],
            out.at[t, pl.ds(base + (NCH - 1) * CH, CH)],
            ssem,
        ).wait()


def kernel(source_weight, target_weight, src_indices, tgt_indices):
    src_i = src_indices.astype(jnp.int32).reshape(NW, NCH, CH)
    tgt_i = tgt_indices.astype(jnp.int32).reshape(NW, NCH, CH)
    out = _emb_lookup(source_weight, target_weight, src_i, tgt_i)
    return out.reshape(2, 16384, 50, D)


# speedup vs baseline: 5.5660x; 5.5660x over previous
import functools

import jax
import jax.numpy as jnp
from jax import lax
from jax.experimental import pallas as pl
from jax.experimental.pallas import tpu as pltpu
from jax.experimental.pallas import tpu_sc as plsc

D = 64
B = 16384 * 50
NW = 32
BPW = B // NW
CH = 128
NCH = BPW // CH
NBUF = 2

_mesh = plsc.VectorSubcoreMesh(core_axis_name="c", subcore_axis_name="s")


@functools.partial(
    pl.kernel,
    mesh=_mesh,
    compiler_params=pltpu.CompilerParams(use_tc_tiling_on_sc=False),
    out_type=jax.ShapeDtypeStruct((2, B, D), jnp.float32),
    scratch_types=[
        pltpu.VMEM((NCH, CH), jnp.int32),
        pltpu.VMEM((NBUF, CH, D), jnp.float32),
        pltpu.SemaphoreType.DMA,
        pltpu.SemaphoreType.DMA,
    ],
)
def _emb_lookup(src_w, tgt_w, src_idx, tgt_idx, out, idx_v, rows_v, gsem, ssem):
    wid = lax.axis_index("s") * 2 + lax.axis_index("c")
    base = wid * BPW
    for t in range(2):
        table = src_w if t == 0 else tgt_w
        idx_hbm = src_idx if t == 0 else tgt_idx
        pltpu.sync_copy(idx_hbm.at[wid], idx_v)

        pltpu.async_copy(table.at[idx_v.at[0]], rows_v.at[0], gsem)

        def chunk_body(j, _):
            slot = lax.rem(j, NBUF)
            nslot = lax.rem(j + 1, NBUF)

            pltpu.make_async_copy(
                table.at[idx_v.at[j]], rows_v.at[slot], gsem
            ).wait()
            pltpu.async_copy(
                rows_v.at[slot], out.at[t, pl.ds(base + j * CH, CH)], ssem
            )

            @pl.when(j >= 1)
            def _():
                pltpu.make_async_copy(
                    rows_v.at[nslot],
                    out.at[t, pl.ds(base + (j - 1) * CH, CH)],
                    ssem,
                ).wait()

            @pl.when(j + 1 < NCH)
            def _():
                pltpu.async_copy(
                    table.at[idx_v.at[j + 1]], rows_v.at[nslot], gsem
                )

            return 0

        lax.fori_loop(0, NCH, chunk_body, 0)
        pltpu.make_async_copy(
            rows_v.at[(NCH - 1) % NBUF],
            out.at[t, pl.ds(base + (NCH - 1) * CH, CH)],
            ssem,
        ).wait()


def kernel(source_weight, target_weight, src_indices, tgt_indices):
    src_i = src_indices.astype(jnp.int32).reshape(NW, NCH, CH)
    tgt_i = tgt_indices.astype(jnp.int32).reshape(NW, NCH, CH)
    out = _emb_lookup(source_weight, target_weight, src_i, tgt_i)
    return out.reshape(2, 16384, 50, D)
